# baseline (speedup 1.0000x reference)
import functools

import jax
import jax.numpy as jnp
from jax import lax
from jax.experimental import pallas as pl
from jax.experimental.pallas import tpu as pltpu
from jax.experimental.pallas import tpu_sc as plsc

_EMB_DIM = 64
_NPOS = 16384
_NNEG = 81920
_NC = 2
_NS = 16
_NW = _NC * _NS
_SUB = 128
_NSUB = 4
_CHUNK = _SUB * _NSUB
_POS_PASSES = _NPOS // (_NW * _CHUNK)
_NEG_PASSES = _NNEG // (_NW * _CHUNK)


def _log_sigmoid(x):
    ax = jnp.abs(x)
    z = jnp.exp(-ax)
    t = z / (z + 2.0)
    t2 = t * t
    p = 1.0 + t2 * (0.33333334 + t2 * (0.2 + t2 * (0.14285715 + t2 * 0.11111111)))
    return jnp.minimum(x, 0.0) - 2.0 * t * p


def _build():
    mesh = plsc.VectorSubcoreMesh(core_axis_name="c", subcore_axis_name="s")

    @functools.partial(
        pl.kernel,
        mesh=mesh,
        compiler_params=pltpu.CompilerParams(
            needs_layout_passes=False, use_tc_tiling_on_sc=False),
        out_type=jax.ShapeDtypeStruct((_NW, 16), jnp.float32),
        scratch_types=[
            pltpu.VMEM((_NSUB, _SUB), jnp.int32),
            pltpu.VMEM((_NSUB, _SUB), jnp.int32),
            pltpu.VMEM((_CHUNK, _EMB_DIM), jnp.float32),
            pltpu.VMEM((_CHUNK, _EMB_DIM), jnp.float32),
            pltpu.VMEM((16,), jnp.float32),
            pltpu.SemaphoreType.DMA,
        ],
    )
    def sgns(pw, pc, nw, nc, table, out, idxw_v, idxc_v, wbuf, cbuf, part_v, sem):
        wid = lax.axis_index("s") * _NC + lax.axis_index("c")
        lane = lax.iota(jnp.int32, 16)

        def run_pass(acc, widx_hbm, cidx_hbm, row_base, sign):
            pltpu.sync_copy(widx_hbm.at[pl.ds(row_base, _NSUB)], idxw_v)
            pltpu.sync_copy(cidx_hbm.at[pl.ds(row_base, _NSUB)], idxc_v)
            copies = []
            for j in range(_NSUB):
                copies.append(pltpu.async_copy(
                    table.at[idxw_v.at[j]], wbuf.at[pl.ds(j * _SUB, _SUB)], sem))
                copies.append(pltpu.async_copy(
                    table.at[idxc_v.at[j]], cbuf.at[pl.ds(j * _SUB, _SUB)], sem))
            for cp in copies:
                cp.wait()

            def group_body(g, acc):
                dots = jnp.zeros((16,), jnp.float32)
                for i in range(16):
                    p = g * 16 + i
                    r = jnp.zeros((16,), jnp.float32)
                    for k in range(_EMB_DIM // 16):
                        wv = wbuf[p, pl.ds(k * 16, 16)]
                        cv = cbuf[p, pl.ds(k * 16, 16)]
                        r = r + wv * cv
                    dots = jnp.where(lane == i, jnp.sum(r), dots)
                return acc + _log_sigmoid(sign * dots)

            return lax.fori_loop(0, _CHUNK // 16, group_body, acc)

        acc = jnp.zeros((16,), jnp.float32)
        for p in range(_POS_PASSES):
            acc = run_pass(acc, pw, pc, wid * (_POS_PASSES * _NSUB) + p * _NSUB, 1.0)
        for p in range(_NEG_PASSES):
            acc = run_pass(acc, nw, nc, wid * (_NEG_PASSES * _NSUB) + p * _NSUB, -1.0)

        part_v[...] = acc
        pltpu.sync_copy(part_v, out.at[wid])

    return sgns


_sgns_cache = []


def _get_sgns():
    if not _sgns_cache:
        _sgns_cache.append(_build())
    return _sgns_cache[0]


def kernel(pos_w_idx, pos_c_idx, neg_w_idx, neg_c_idx, W, C):
    pw = pos_w_idx.astype(jnp.int32).reshape(_NPOS // _SUB, _SUB)
    pc = pos_c_idx.astype(jnp.int32).reshape(_NPOS // _SUB, _SUB)
    nw = neg_w_idx.astype(jnp.int32).reshape(_NNEG // _SUB, _SUB)
    nc = neg_c_idx.astype(jnp.int32).reshape(_NNEG // _SUB, _SUB)
    partials = _get_sgns()(pw, pc, nw, nc, W)
    return -jnp.sum(partials)
